# speedup vs baseline: 11.6773x; 11.6773x over previous
import jax
import jax.numpy as jnp
from jax.experimental import pallas as pl

N = 10000
DIN = 128
FC = 256
G = 64


def _pool_kernel(x_ref, batch_ref, w_ref, b_ref, out_ref):
    x = x_ref[...]
    b2 = batch_ref[...]
    ids = jax.lax.broadcasted_iota(jnp.int32, (1, G), 1)
    seg = (b2 == ids).astype(jnp.float32)
    sums = jax.lax.dot_general(seg, x, (((0,), (0,)), ((), ())),
                               preferred_element_type=jnp.float32)
    cnt = jnp.sum(seg, axis=0, keepdims=True)
    pooled = sums / jnp.maximum(cnt, 1.0).T
    out_ref[...] = jnp.dot(pooled, w_ref[...],
                           preferred_element_type=jnp.float32) + b_ref[...]


def kernel(x, edge_index, edge_attr, batch, W_atom, b_atom, W_edge, b_edge):
    del edge_index, edge_attr, W_edge, b_edge
    batch2d = batch.reshape(N, 1)
    bias2d = b_atom.reshape(1, FC)
    return pl.pallas_call(
        _pool_kernel,
        out_shape=jax.ShapeDtypeStruct((G, FC), jnp.float32),
    )(x, batch2d, W_atom, bias2d)
